# baseline (speedup 1.0000x reference)
import functools
import math

import jax
import jax.numpy as jnp
from jax import lax
from jax.experimental import pallas as pl
from jax.experimental.pallas import tpu as pltpu
from jax.experimental.pallas import tpu_sc as plsc

_EMBED = 128
_LANES = 16
_NC, _NS = 2, 16
_NW = _NC * _NS
_K = 128
_VPAD = 6272
_SLOTS = 4


@functools.cache
def _build(n_idx, vocab):
    per_w = n_idx // _NW
    nchunks = per_w // _K
    total_j = nchunks + _SLOTS
    stage_chunks = _VPAD // _K
    scale = jnp.float32(math.sqrt(_EMBED))
    mesh = plsc.VectorSubcoreMesh(
        core_axis_name="c", subcore_axis_name="s",
        num_cores=_NC, num_subcores=_NS)

    @functools.partial(
        pl.kernel,
        out_type=(
            jax.ShapeDtypeStruct((n_idx, _EMBED), jnp.float32),
            jax.ShapeDtypeStruct((_NC * _VPAD, _EMBED), jnp.float32),
        ),
        mesh=mesh,
        scratch_types=[
            pltpu.VMEM_SHARED((_VPAD, _EMBED), jnp.float32),
            [pltpu.VMEM((_K,), jnp.int32) for _ in range(_SLOTS)],
            [pltpu.VMEM((_K,), jnp.int32) for _ in range(_SLOTS)],
            [pltpu.VMEM((_K, _EMBED), jnp.float32) for _ in range(_SLOTS)],
            [pltpu.SemaphoreType.DMA for _ in range(_SLOTS)],
            [pltpu.SemaphoreType.DMA for _ in range(_SLOTS)],
            [pltpu.SemaphoreType.DMA for _ in range(_SLOTS)],
            [pltpu.SemaphoreType.DMA for _ in range(_SLOTS)],
        ],
    )
    def emb_kernel(x1_hbm, x2_hbm, lut_hbm, out_hbm, neg_hbm, table_sh,
                   idx1, idx2, rows, isem, g2sem, g1sem, osem):
        cid = lax.axis_index("c")
        sid = lax.axis_index("s")
        wid = sid * _NC + cid
        base = wid * per_w
        row0 = wid * nchunks
        neg_base = cid * _VPAD

        def stage(s, _):
            c0 = s * _NS + sid

            @pl.when(c0 < stage_chunks)
            def _():
                off = c0 * _K
                buf = rows[0]
                pltpu.sync_copy(lut_hbm.at[pl.ds(off, _K)], buf)

                @plsc.parallel_loop(0, _K, unroll=2)
                def _row(r):
                    for c in range(_EMBED // _LANES):
                        sl = pl.ds(c * _LANES, _LANES)
                        buf[r, sl] = buf[r, sl] * scale

                pltpu.sync_copy(buf, table_sh.at[pl.ds(off, _K)])

                @plsc.parallel_loop(0, _K, unroll=2)
                def _row(r):
                    for c in range(_EMBED // _LANES):
                        sl = pl.ds(c * _LANES, _LANES)
                        buf[r, sl] = -buf[r, sl]

                pltpu.sync_copy(buf, neg_hbm.at[pl.ds(neg_base + off, _K)])

            return 0

        lax.fori_loop(0, (stage_chunks + _NS - 1) // _NS, stage, 0)
        plsc.subcore_barrier()

        def istart(j, b):
            pltpu.async_copy(x1_hbm.at[row0 + j], idx1[b], isem[b])
            pltpu.async_copy(x2_hbm.at[row0 + j], idx2[b], isem[b])

        def iwait(b):
            pltpu.make_async_copy(x1_hbm.at[0], idx1[b], isem[b]).wait()
            pltpu.make_async_copy(x1_hbm.at[0], idx2[b], isem[b]).wait()
            @plsc.parallel_loop(0, _K // _LANES)
            def _v(v):
                sl = pl.ds(v * _LANES, _LANES)
                idx1[b][sl] = idx1[b][sl] + neg_base

        def g2start(b):
            pltpu.async_copy(table_sh.at[idx2[b]], rows[b], g2sem[b])

        def g2wait(b):
            pltpu.make_async_copy(table_sh.at[pl.ds(0, _K)], rows[b], g2sem[b]).wait()

        def g1start(b):
            pltpu.async_copy(neg_hbm.at[idx1[b]], rows[b], g1sem[b], add=True)

        def g1wait(b):
            pltpu.make_async_copy(neg_hbm.at[pl.ds(0, _K)], rows[b], g1sem[b]).wait()

        def ostart(j, b):
            pltpu.async_copy(rows[b], out_hbm.at[pl.ds(base + j * _K, _K)], osem[b])

        def owait(b):
            pltpu.make_async_copy(rows[b], out_hbm.at[pl.ds(0, _K)], osem[b]).wait()

        istart(0, 0)
        istart(1, 1)

        def outer(jo, _):
            for bb in range(_SLOTS):
                j = _SLOTS * jo + bb
                bm2 = (bb - 2) % _SLOTS
                bm1 = (bb - 1) % _SLOTS
                bp2 = (bb + 2) % _SLOTS

                @pl.when((j >= 2) & (j - 2 < nchunks))
                def _():
                    g1wait(bm2)
                    ostart(j - 2, bm2)

                @pl.when(j + 2 < nchunks)
                def _():
                    istart(j + 2, bp2)

                @pl.when(j < nchunks)
                def _():
                    @pl.when(j >= _SLOTS)
                    def _():
                        owait(bb)

                    iwait(bb)
                    g2start(bb)

                @pl.when((j >= 1) & (j - 1 < nchunks))
                def _():
                    g2wait(bm1)
                    g1start(bm1)

            return 0

        lax.fori_loop(0, total_j // _SLOTS, outer, 0)
        for bb in range(_SLOTS):
            owait(bb)

    return emb_kernel


def kernel(x1, x2, lut):
    b, t = x1.shape
    n_idx = b * t
    x1f = x1.reshape(n_idx // _K, _K).astype(jnp.int32)
    x2f = x2.reshape(n_idx // _K, _K).astype(jnp.int32)
    lut_pad = jnp.pad(lut, ((0, _VPAD - lut.shape[0]), (0, 0)))
    out, _ = _build(n_idx, lut.shape[0])(x1f, x2f, lut_pad)
    return out.reshape(b, t, _EMBED)
